# baseline (speedup 1.0000x reference)
import functools

import jax
import jax.numpy as jnp
from jax import lax
from jax.experimental import pallas as pl
from jax.experimental.pallas import tpu as pltpu
from jax.experimental.pallas import tpu_sc as plsc

BATCH = 64
NUM_TX = 4
NUM_STREAMS = 2
NUM_OFDM = 14
FFT = 4096
NUM_DATA = 12
PAIRS = BATCH * NUM_TX * NUM_STREAMS
NUM_WORKERS = 32
PAIRS_PER_W = PAIRS // NUM_WORKERS
IN_ROW = NUM_DATA * FFT
OUT_ROW = NUM_OFDM * FFT


def _sc_grid_map(x1, p1):
    mesh = plsc.VectorSubcoreMesh(core_axis_name="c", subcore_axis_name="s")

    @functools.partial(
        pl.kernel,
        mesh=mesh,
        out_type=jax.ShapeDtypeStruct((PAIRS * OUT_ROW,), jnp.float32),
        scratch_types=[
            pltpu.SemaphoreType.DMA,
        ],
    )
    def grid_map(x_hbm, p_hbm, out_hbm, sem):
        wid = lax.axis_index("s") * 2 + lax.axis_index("c")
        base = wid * PAIRS_PER_W
        waits = []
        for j in range(PAIRS_PER_W):
            xb = pl.multiple_of((base + j) * IN_ROW, FFT)
            ob = pl.multiple_of((base + j) * OUT_ROW, FFT)
            q = j % (NUM_TX * NUM_STREAMS)
            waits += [
                pltpu.async_copy(x_hbm.at[pl.ds(xb, 2 * FFT)], out_hbm.at[pl.ds(ob, 2 * FFT)], sem),
                pltpu.async_copy(x_hbm.at[pl.ds(xb + 2 * FFT, 8 * FFT)], out_hbm.at[pl.ds(ob + 3 * FFT, 8 * FFT)], sem),
                pltpu.async_copy(x_hbm.at[pl.ds(xb + 10 * FFT, 2 * FFT)], out_hbm.at[pl.ds(ob + 12 * FFT, 2 * FFT)], sem),
                pltpu.async_copy(p_hbm.at[pl.ds(q * 2 * FFT, FFT)], out_hbm.at[pl.ds(ob + 2 * FFT, FFT)], sem),
                pltpu.async_copy(p_hbm.at[pl.ds(q * 2 * FFT + FFT, FFT)], out_hbm.at[pl.ds(ob + 11 * FFT, FFT)], sem),
            ]
        for w in waits:
            w.wait()

    return grid_map(x1, p1)


def kernel(x, pilots):
    rg = _sc_grid_map(x.reshape(-1), pilots.reshape(-1))
    return rg.reshape(BATCH, NUM_TX, NUM_STREAMS, NUM_OFDM, FFT)


# speedup vs baseline: 9.2253x; 9.2253x over previous
import functools

import jax
import jax.numpy as jnp
from jax import lax
from jax.experimental import pallas as pl
from jax.experimental.pallas import tpu as pltpu
from jax.experimental.pallas import tpu_sc as plsc

BATCH = 64
NUM_TX = 4
NUM_STREAMS = 2
NUM_OFDM = 14
FFT = 4096
NUM_DATA = 12
PAIRS = BATCH * NUM_TX * NUM_STREAMS
NUM_WORKERS = 32
PAIRS_PER_W = PAIRS // NUM_WORKERS
IN_ROW = NUM_DATA * FFT
OUT_ROW = NUM_OFDM * FFT


def _sc_grid_map(x1, p1):
    mesh = plsc.VectorSubcoreMesh(core_axis_name="c", subcore_axis_name="s")

    @functools.partial(
        pl.kernel,
        mesh=mesh,
        out_type=jax.ShapeDtypeStruct((PAIRS * OUT_ROW,), jnp.float32),
        scratch_types=[
            pltpu.VMEM((OUT_ROW,), jnp.float32),
            pltpu.VMEM((OUT_ROW,), jnp.float32),
            pltpu.SemaphoreType.DMA,
            pltpu.SemaphoreType.DMA,
            pltpu.SemaphoreType.DMA,
            pltpu.SemaphoreType.DMA,
        ],
    )
    def grid_map(x_hbm, p_hbm, out_hbm, buf0, buf1, in0, in1, out0, out1):
        wid = lax.axis_index("s") * 2 + lax.axis_index("c")
        base = wid * PAIRS_PER_W
        bufs = (buf0, buf1)
        in_sems = (in0, in1)
        out_sems = (out0, out1)
        out_waits = [None, None]
        for j in range(PAIRS_PER_W):
            slot = j % 2
            buf = bufs[slot]
            sem = in_sems[slot]
            xb = pl.multiple_of((base + j) * IN_ROW, FFT)
            ob = pl.multiple_of((base + j) * OUT_ROW, FFT)
            q = j % (NUM_TX * NUM_STREAMS)
            if out_waits[slot] is not None:
                out_waits[slot].wait()
            copies = (
                pltpu.async_copy(x_hbm.at[pl.ds(xb, 2 * FFT)], buf.at[pl.ds(0, 2 * FFT)], sem),
                pltpu.async_copy(x_hbm.at[pl.ds(xb + 2 * FFT, 8 * FFT)], buf.at[pl.ds(3 * FFT, 8 * FFT)], sem),
                pltpu.async_copy(x_hbm.at[pl.ds(xb + 10 * FFT, 2 * FFT)], buf.at[pl.ds(12 * FFT, 2 * FFT)], sem),
                pltpu.async_copy(p_hbm.at[pl.ds(q * 2 * FFT, FFT)], buf.at[pl.ds(2 * FFT, FFT)], sem),
                pltpu.async_copy(p_hbm.at[pl.ds(q * 2 * FFT + FFT, FFT)], buf.at[pl.ds(11 * FFT, FFT)], sem),
            )
            for c in copies:
                c.wait()
            out_waits[slot] = pltpu.async_copy(buf, out_hbm.at[pl.ds(ob, OUT_ROW)], out_sems[slot])
        for w in out_waits:
            w.wait()

    return grid_map(x1, p1)


def kernel(x, pilots):
    rg = _sc_grid_map(x.reshape(-1), pilots.reshape(-1))
    return rg.reshape(BATCH, NUM_TX, NUM_STREAMS, NUM_OFDM, FFT)


# speedup vs baseline: 33.6130x; 3.6436x over previous
import functools

import jax
import jax.numpy as jnp
from jax import lax
from jax.experimental import pallas as pl
from jax.experimental.pallas import tpu as pltpu
from jax.experimental.pallas import tpu_sc as plsc

BATCH = 64
NUM_TX = 4
NUM_STREAMS = 2
NUM_OFDM = 14
FFT = 4096
NUM_DATA = 12
SLABS = BATCH * NUM_TX
NUM_WORKERS = 32
HALVES_PER_W = 2 * SLABS // NUM_WORKERS
BLK = NUM_STREAMS * FFT
X_SLAB = NUM_DATA * BLK
O_SLAB = NUM_OFDM * BLK
HALF = 7 * BLK


def _sc_grid_map(x_lin, p_lin):
    mesh = plsc.VectorSubcoreMesh(core_axis_name="c", subcore_axis_name="s")

    @functools.partial(
        pl.kernel,
        mesh=mesh,
        out_type=jax.ShapeDtypeStruct((SLABS * O_SLAB,), jnp.float32),
        scratch_types=[
            pltpu.VMEM((HALF,), jnp.float32),
            pltpu.VMEM((HALF,), jnp.float32),
            pltpu.SemaphoreType.DMA,
            pltpu.SemaphoreType.DMA,
            pltpu.SemaphoreType.DMA,
            pltpu.SemaphoreType.DMA,
        ],
    )
    def grid_map(x_hbm, p_hbm, out_hbm, buf0, buf1, in0, in1, out0, out1):
        wid = lax.axis_index("s") * 2 + lax.axis_index("c")
        bufs = (buf0, buf1)
        in_sems = (in0, in1)
        out_sems = (out0, out1)
        out_waits = [None, None]
        for j in range(HALVES_PER_W):
            slot = j % 2
            buf = bufs[slot]
            sem = in_sems[slot]
            half = j % 2
            txq = (j // 2) % NUM_TX
            bt = 8 * wid + j // 2
            xb = pl.multiple_of(bt * X_SLAB, BLK)
            ob = pl.multiple_of(bt * O_SLAB + half * HALF, BLK)
            if out_waits[slot] is not None:
                out_waits[slot].wait()
            if half == 0:
                copies = (
                    pltpu.async_copy(x_hbm.at[pl.ds(xb, 2 * BLK)], buf.at[pl.ds(0, 2 * BLK)], sem),
                    pltpu.async_copy(p_hbm.at[pl.ds((txq * 2) * BLK, BLK)], buf.at[pl.ds(2 * BLK, BLK)], sem),
                    pltpu.async_copy(x_hbm.at[pl.ds(xb + 2 * BLK, 4 * BLK)], buf.at[pl.ds(3 * BLK, 4 * BLK)], sem),
                )
            else:
                copies = (
                    pltpu.async_copy(x_hbm.at[pl.ds(xb + 6 * BLK, 4 * BLK)], buf.at[pl.ds(0, 4 * BLK)], sem),
                    pltpu.async_copy(p_hbm.at[pl.ds((txq * 2 + 1) * BLK, BLK)], buf.at[pl.ds(4 * BLK, BLK)], sem),
                    pltpu.async_copy(x_hbm.at[pl.ds(xb + 10 * BLK, 2 * BLK)], buf.at[pl.ds(5 * BLK, 2 * BLK)], sem),
                )
            for c in copies:
                c.wait()
            out_waits[slot] = pltpu.async_copy(buf, out_hbm.at[pl.ds(ob, HALF)], out_sems[slot])
        for w in out_waits:
            w.wait()

    return grid_map(x_lin, p_lin)


def kernel(x, pilots):
    x_lin = (
        x.reshape(SLABS, NUM_STREAMS, NUM_DATA * 32, 128)
        .transpose(0, 2, 1, 3)
        .reshape(-1)
    )
    p_lin = (
        pilots.reshape(NUM_TX, NUM_STREAMS, 2, 32, 128)
        .transpose(0, 2, 3, 1, 4)
        .reshape(-1)
    )
    o_lin = _sc_grid_map(x_lin, p_lin)
    return (
        o_lin.reshape(SLABS, NUM_OFDM, 32, NUM_STREAMS, 128)
        .transpose(0, 3, 1, 2, 4)
        .reshape(BATCH, NUM_TX, NUM_STREAMS, NUM_OFDM, FFT)
    )


# speedup vs baseline: 38.8929x; 1.1571x over previous
import functools

import jax
import jax.numpy as jnp
from jax import lax
from jax.experimental import pallas as pl
from jax.experimental.pallas import tpu as pltpu
from jax.experimental.pallas import tpu_sc as plsc

BATCH = 64
NUM_TX = 4
NUM_STREAMS = 2
NUM_OFDM = 14
FFT = 4096
NUM_DATA = 12
SLABS = BATCH * NUM_TX
NUM_WORKERS = 32
HALVES_PER_W = 2 * SLABS // NUM_WORKERS
BLK = NUM_STREAMS * FFT
X_SLAB = NUM_DATA * BLK
O_SLAB = NUM_OFDM * BLK
HALF = 7 * BLK


def _sc_grid_map(x_lin, p_lin):
    mesh = plsc.VectorSubcoreMesh(core_axis_name="c", subcore_axis_name="s")

    @functools.partial(
        pl.kernel,
        mesh=mesh,
        out_type=jax.ShapeDtypeStruct((SLABS * O_SLAB,), jnp.float32),
        scratch_types=[
            pltpu.VMEM((6 * BLK,), jnp.float32),
            pltpu.VMEM((6 * BLK,), jnp.float32),
            pltpu.VMEM((2 * BLK,), jnp.float32),
            pltpu.SemaphoreType.DMA,
            pltpu.SemaphoreType.DMA,
            pltpu.SemaphoreType.DMA,
            pltpu.SemaphoreType.DMA,
        ],
    )
    def grid_map(x_hbm, p_hbm, out_hbm, buf0, buf1, pbuf, in0, in1, out0, out1):
        wid = lax.axis_index("s") * 2 + lax.axis_index("c")
        tx = wid % NUM_TX
        bt0 = 32 * (wid // NUM_TX) + tx
        pltpu.async_copy(
            p_hbm.at[pl.ds(pl.multiple_of(tx * 2 * BLK, BLK), 2 * BLK)], pbuf, in0
        ).wait()
        bufs = (buf0, buf1)
        in_sems = (in0, in1)
        out_sems = (out0, out1)
        out_waits = [None, None]
        for j in range(HALVES_PER_W):
            slot = j % 2
            buf = bufs[slot]
            half = j % 2
            bt = bt0 + NUM_TX * (j // 2)
            xb = pl.multiple_of(bt * X_SLAB + half * 6 * BLK, BLK)
            ob = pl.multiple_of(bt * O_SLAB + half * HALF, BLK)
            if out_waits[slot] is not None:
                for w in out_waits[slot]:
                    w.wait()
            pltpu.async_copy(x_hbm.at[pl.ds(xb, 6 * BLK)], buf, in_sems[slot]).wait()
            osem = out_sems[slot]
            if half == 0:
                out_waits[slot] = (
                    pltpu.async_copy(buf.at[pl.ds(0, 2 * BLK)], out_hbm.at[pl.ds(ob, 2 * BLK)], osem),
                    pltpu.async_copy(pbuf.at[pl.ds(0, BLK)], out_hbm.at[pl.ds(ob + 2 * BLK, BLK)], osem),
                    pltpu.async_copy(buf.at[pl.ds(2 * BLK, 4 * BLK)], out_hbm.at[pl.ds(ob + 3 * BLK, 4 * BLK)], osem),
                )
            else:
                out_waits[slot] = (
                    pltpu.async_copy(buf.at[pl.ds(0, 4 * BLK)], out_hbm.at[pl.ds(ob, 4 * BLK)], osem),
                    pltpu.async_copy(pbuf.at[pl.ds(BLK, BLK)], out_hbm.at[pl.ds(ob + 4 * BLK, BLK)], osem),
                    pltpu.async_copy(buf.at[pl.ds(4 * BLK, 2 * BLK)], out_hbm.at[pl.ds(ob + 5 * BLK, 2 * BLK)], osem),
                )
        for ws in out_waits:
            for w in ws:
                w.wait()

    return grid_map(x_lin, p_lin)


def kernel(x, pilots):
    x_lin = (
        x.reshape(SLABS, NUM_STREAMS, NUM_DATA * 32, 128)
        .transpose(0, 2, 1, 3)
        .reshape(-1)
    )
    p_lin = (
        pilots.reshape(NUM_TX, NUM_STREAMS, 2, 32, 128)
        .transpose(0, 2, 3, 1, 4)
        .reshape(-1)
    )
    o_lin = _sc_grid_map(x_lin, p_lin)
    return (
        o_lin.reshape(SLABS, NUM_OFDM, 32, NUM_STREAMS, 128)
        .transpose(0, 3, 1, 2, 4)
        .reshape(BATCH, NUM_TX, NUM_STREAMS, NUM_OFDM, FFT)
    )
